# Initial kernel scaffold; baseline (speedup 1.0000x reference)
import jax
import jax.numpy as jnp
from jax.experimental import pallas as pl


def kernel(x, edge_index, W, a_src, a_dst, b):
    raise NotImplementedError("write your pallas kernel here")


# speedup vs baseline: 1.1912x; 1.1912x over previous
import jax
import jax.numpy as jnp
from jax.experimental import pallas as pl


def _proj_kernel(x_ref, w_ref, h_ref):
    h_ref[...] = jnp.dot(x_ref[...], w_ref[...], preferred_element_type=jnp.float32)


def _attn_kernel(h_ref, asrc_ref, adst_ref, si_ref, di_ref, sj_ref, dj_ref,
                 b_ref, out_ref):
    i = pl.program_id(0)
    blk = out_ref.shape[0]
    h = h_ref[...]
    hb = h_ref[pl.ds(i * blk, blk), :]
    ed = jnp.dot(hb, adst_ref[...], preferred_element_type=jnp.float32)
    es = jax.lax.dot_general(asrc_ref[...], h, (((1,), (1,)), ((), ())),
                             preferred_element_type=jnp.float32)
    logits = ed + es
    logits = jnp.where(logits > 0, logits, 0.2 * logits)
    si = si_ref[...]
    di = di_ref[...]
    sj = sj_ref[...]
    dj = dj_ref[...]
    conn = (si == sj) | (si == dj) | (di == sj) | (di == dj)
    logits = jnp.where(conn, logits, jnp.float32(-1e30))
    m = jnp.max(logits, axis=1, keepdims=True)
    p = jnp.where(conn, jnp.exp(logits - m), jnp.float32(0.0))
    s = jnp.sum(p, axis=1, keepdims=True)
    alpha = p / s
    out_ref[...] = (jnp.dot(alpha, h, preferred_element_type=jnp.float32)
                    + b_ref[...])


def kernel(x, edge_index, W, a_src, a_dst, b):
    E, _ = x.shape
    C = W.shape[1]
    h = pl.pallas_call(
        _proj_kernel,
        out_shape=jax.ShapeDtypeStruct((E, C), jnp.float32),
    )(x, W)

    blk = 512
    src = edge_index[0]
    dst = edge_index[1]
    out = pl.pallas_call(
        _attn_kernel,
        grid=(E // blk,),
        in_specs=[
            pl.BlockSpec((E, C), lambda i: (0, 0)),
            pl.BlockSpec((1, C), lambda i: (0, 0)),
            pl.BlockSpec((C, 1), lambda i: (0, 0)),
            pl.BlockSpec((blk, 1), lambda i: (i, 0)),
            pl.BlockSpec((blk, 1), lambda i: (i, 0)),
            pl.BlockSpec((1, E), lambda i: (0, 0)),
            pl.BlockSpec((1, E), lambda i: (0, 0)),
            pl.BlockSpec((1, C), lambda i: (0, 0)),
        ],
        out_specs=pl.BlockSpec((blk, C), lambda i: (i, 0)),
        out_shape=jax.ShapeDtypeStruct((E, C), jnp.float32),
    )(h, a_src.reshape(1, C), a_dst.reshape(C, 1),
      src.reshape(E, 1), dst.reshape(E, 1),
      src.reshape(1, E), dst.reshape(1, E),
      b.reshape(1, C))
    return out


# speedup vs baseline: 1.3400x; 1.1249x over previous
import jax
import jax.numpy as jnp
from jax.experimental import pallas as pl


def _proj_kernel(x_ref, w_ref, h_ref):
    h_ref[...] = jnp.dot(x_ref[...], w_ref[...], preferred_element_type=jnp.float32)


def _attn_kernel(h_ref, asrc_ref, adst_ref, si_ref, di_ref, sj_ref, dj_ref,
                 b_ref, out_ref):
    i = pl.program_id(0)
    blk = out_ref.shape[0]
    h = h_ref[...]
    hb = h_ref[pl.ds(i * blk, blk), :]
    ed = jnp.dot(hb, adst_ref[...], preferred_element_type=jnp.float32)
    es = jax.lax.dot_general(asrc_ref[...], h, (((1,), (1,)), ((), ())),
                             preferred_element_type=jnp.float32)
    smax = jnp.max(es)
    mi = ed + smax
    mi = jnp.maximum(mi, 0.2 * mi)
    z = ed + es
    z = jnp.maximum(z, 0.2 * z)
    si = si_ref[...]
    di = di_ref[...]
    sj = sj_ref[...]
    dj = dj_ref[...]
    prod = ((si - sj) * (si - dj)) * ((di - sj) * (di - dj))
    p = jnp.where(prod == 0.0, jnp.exp(z - mi), jnp.float32(0.0))
    s = jnp.sum(p, axis=1, keepdims=True)
    acc = jnp.dot(p, h, preferred_element_type=jnp.float32)
    out_ref[...] = acc / s + b_ref[...]


def kernel(x, edge_index, W, a_src, a_dst, b):
    E, _ = x.shape
    C = W.shape[1]
    h = pl.pallas_call(
        _proj_kernel,
        out_shape=jax.ShapeDtypeStruct((E, C), jnp.float32),
    )(x, W)

    blk = 512
    srcf = edge_index[0].astype(jnp.float32)
    dstf = edge_index[1].astype(jnp.float32)
    out = pl.pallas_call(
        _attn_kernel,
        grid=(E // blk,),
        in_specs=[
            pl.BlockSpec((E, C), lambda i: (0, 0)),
            pl.BlockSpec((1, C), lambda i: (0, 0)),
            pl.BlockSpec((C, 1), lambda i: (0, 0)),
            pl.BlockSpec((blk, 1), lambda i: (i, 0)),
            pl.BlockSpec((blk, 1), lambda i: (i, 0)),
            pl.BlockSpec((1, E), lambda i: (0, 0)),
            pl.BlockSpec((1, E), lambda i: (0, 0)),
            pl.BlockSpec((1, C), lambda i: (0, 0)),
        ],
        out_specs=pl.BlockSpec((blk, C), lambda i: (i, 0)),
        out_shape=jax.ShapeDtypeStruct((E, C), jnp.float32),
    )(h, a_src.reshape(1, C), a_dst.reshape(C, 1),
      srcf.reshape(E, 1), dstf.reshape(E, 1),
      srcf.reshape(1, E), dstf.reshape(1, E),
      b.reshape(1, C))
    return out


# speedup vs baseline: 1.3525x; 1.0093x over previous
import jax
import jax.numpy as jnp
from jax.experimental import pallas as pl


def _proj_kernel(x_ref, w_ref, h_ref):
    h_ref[...] = jnp.dot(x_ref[...], w_ref[...], preferred_element_type=jnp.float32)


def _attn_kernel(h_ref, asrc_ref, adst_ref, si_ref, di_ref, sj_ref, dj_ref,
                 b_ref, out_ref):
    i = pl.program_id(0)
    blk = out_ref.shape[0]
    h = h_ref[...]
    hb = h_ref[pl.ds(i * blk, blk), :]
    ed = jnp.dot(hb, adst_ref[...], preferred_element_type=jnp.float32)
    es = jax.lax.dot_general(asrc_ref[...], h, (((1,), (1,)), ((), ())),
                             preferred_element_type=jnp.float32)
    smax = jnp.max(es)
    q = ed + smax
    mi = jnp.maximum(q, 0.2 * q)
    edm = ed - mi
    c2 = 0.2 * ed - mi
    es02 = 0.2 * es
    sj = sj_ref[...]
    dj = dj_ref[...]
    u = sj + dj
    v = sj * dj
    si = si_ref[...]
    di = di_ref[...]
    si2 = si * si
    di2 = di * di
    zs = jnp.maximum(edm + es, c2 + es02)
    pa = (si2 + v) - si * u
    pb = (di2 + v) - di * u
    p = jnp.where(pa * pb == 0.0, jnp.exp(zs), jnp.float32(0.0))
    s = jnp.sum(p, axis=1, keepdims=True)
    acc = jnp.dot(p.astype(jnp.bfloat16), h.astype(jnp.bfloat16),
                  preferred_element_type=jnp.float32)
    out_ref[...] = acc * (1.0 / s) + b_ref[...]


def kernel(x, edge_index, W, a_src, a_dst, b):
    E, _ = x.shape
    C = W.shape[1]
    h = pl.pallas_call(
        _proj_kernel,
        out_shape=jax.ShapeDtypeStruct((E, C), jnp.float32),
    )(x, W)

    blk = 512
    srcf = edge_index[0].astype(jnp.float32)
    dstf = edge_index[1].astype(jnp.float32)
    out = pl.pallas_call(
        _attn_kernel,
        grid=(E // blk,),
        in_specs=[
            pl.BlockSpec((E, C), lambda i: (0, 0)),
            pl.BlockSpec((1, C), lambda i: (0, 0)),
            pl.BlockSpec((C, 1), lambda i: (0, 0)),
            pl.BlockSpec((blk, 1), lambda i: (i, 0)),
            pl.BlockSpec((blk, 1), lambda i: (i, 0)),
            pl.BlockSpec((1, E), lambda i: (0, 0)),
            pl.BlockSpec((1, E), lambda i: (0, 0)),
            pl.BlockSpec((1, C), lambda i: (0, 0)),
        ],
        out_specs=pl.BlockSpec((blk, C), lambda i: (i, 0)),
        out_shape=jax.ShapeDtypeStruct((E, C), jnp.float32),
    )(h, a_src.reshape(1, C), a_dst.reshape(C, 1),
      srcf.reshape(E, 1), dstf.reshape(E, 1),
      srcf.reshape(1, E), dstf.reshape(1, E),
      b.reshape(1, C))
    return out


# speedup vs baseline: 1.5658x; 1.1577x over previous
import jax
import jax.numpy as jnp
from jax.experimental import pallas as pl
from jax.experimental.pallas import tpu as pltpu

_LOG2E = 1.4426950408889634


def _gat_kernel(x_ref, w_ref, asrc_ref, adst_ref, si_ref, di_ref, sj_ref,
                dj_ref, b_ref, out_ref, h_ref, hbf_ref, es2_ref, es022_ref,
                u_ref, v_ref):
    i = pl.program_id(0)
    blk = out_ref.shape[0]

    @pl.when(i == 0)
    def _init():
        h = jnp.dot(x_ref[...], w_ref[...], preferred_element_type=jnp.float32)
        h_ref[...] = h
        hbf_ref[...] = h.astype(jnp.bfloat16)
        es = jax.lax.dot_general(asrc_ref[...], h, (((1,), (1,)), ((), ())),
                                 preferred_element_type=jnp.float32)
        es2_ref[...] = _LOG2E * es
        es022_ref[...] = (0.2 * _LOG2E) * es
        sj = sj_ref[...]
        dj = dj_ref[...]
        u_ref[...] = sj + dj
        v_ref[...] = sj * dj

    hb = h_ref[pl.ds(i * blk, blk), :]
    ed = jnp.dot(hb, adst_ref[...], preferred_element_type=jnp.float32)
    es2 = es2_ref[...]
    smax2 = jnp.max(es2)
    q2 = _LOG2E * ed + smax2
    mi2 = jnp.maximum(q2, 0.2 * q2)
    edm2 = _LOG2E * ed - mi2
    c22 = (0.2 * _LOG2E) * ed - mi2
    si = si_ref[...]
    di = di_ref[...]
    si2 = si * si
    di2 = di * di
    u = u_ref[...]
    v = v_ref[...]
    zs = jnp.maximum(edm2 + es2, c22 + es022_ref[...])
    pa = (si2 + v) - si * u
    pb = (di2 + v) - di * u
    p = jnp.where(pa * pb == 0.0, jnp.exp2(zs), jnp.float32(0.0))
    s = jnp.sum(p, axis=1, keepdims=True)
    acc = jnp.dot(p.astype(jnp.bfloat16), hbf_ref[...],
                  preferred_element_type=jnp.float32)
    out_ref[...] = acc * (1.0 / s) + b_ref[...]


def kernel(x, edge_index, W, a_src, a_dst, b):
    E, _ = x.shape
    C = W.shape[1]
    blk = 512
    srcf = edge_index[0].astype(jnp.float32)
    dstf = edge_index[1].astype(jnp.float32)
    out = pl.pallas_call(
        _gat_kernel,
        grid=(E // blk,),
        in_specs=[
            pl.BlockSpec((E, x.shape[1]), lambda i: (0, 0)),
            pl.BlockSpec((x.shape[1], C), lambda i: (0, 0)),
            pl.BlockSpec((1, C), lambda i: (0, 0)),
            pl.BlockSpec((C, 1), lambda i: (0, 0)),
            pl.BlockSpec((blk, 1), lambda i: (i, 0)),
            pl.BlockSpec((blk, 1), lambda i: (i, 0)),
            pl.BlockSpec((1, E), lambda i: (0, 0)),
            pl.BlockSpec((1, E), lambda i: (0, 0)),
            pl.BlockSpec((1, C), lambda i: (0, 0)),
        ],
        out_specs=pl.BlockSpec((blk, C), lambda i: (i, 0)),
        out_shape=jax.ShapeDtypeStruct((E, C), jnp.float32),
        scratch_shapes=[
            pltpu.VMEM((E, C), jnp.float32),
            pltpu.VMEM((E, C), jnp.bfloat16),
            pltpu.VMEM((1, E), jnp.float32),
            pltpu.VMEM((1, E), jnp.float32),
            pltpu.VMEM((1, E), jnp.float32),
            pltpu.VMEM((1, E), jnp.float32),
        ],
    )(x, W, a_src.reshape(1, C), a_dst.reshape(C, 1),
      srcf.reshape(E, 1), dstf.reshape(E, 1),
      srcf.reshape(1, E), dstf.reshape(1, E),
      b.reshape(1, C))
    return out


# speedup vs baseline: 2.0462x; 1.3068x over previous
import jax
import jax.numpy as jnp
from jax.experimental import pallas as pl
from jax.experimental.pallas import tpu as pltpu

_LOG2E = 1.4426950408889634


def _gat_kernel(x_ref, w_ref, asrc_ref, adst_ref, si_ref, di_ref, sj_ref,
                dj_ref, b_ref, out_ref, h_ref, hbf_ref, es2_ref, es022_ref):
    i = pl.program_id(0)
    blk = out_ref.shape[0]
    C = out_ref.shape[1]

    @pl.when(i == 0)
    def _init():
        h = jnp.dot(x_ref[...], w_ref[...], preferred_element_type=jnp.float32)
        h_ref[...] = h
        hbf_ref[...] = jnp.concatenate(
            [h, jnp.ones((h.shape[0], 128), jnp.float32)], axis=1
        ).astype(jnp.bfloat16)
        es = jax.lax.dot_general(asrc_ref[...], h, (((1,), (1,)), ((), ())),
                                 preferred_element_type=jnp.float32)
        es2_ref[...] = _LOG2E * es
        es022_ref[...] = (0.2 * _LOG2E) * es

    hb = h_ref[pl.ds(i * blk, blk), :]
    ed = jnp.dot(hb, adst_ref[...], preferred_element_type=jnp.float32)
    es2 = es2_ref[...]
    smax2 = jnp.max(es2)
    q2 = _LOG2E * ed + smax2
    mi2 = jnp.maximum(q2, 0.2 * q2)
    edm2 = _LOG2E * ed - mi2
    c22 = (0.2 * _LOG2E) * ed - mi2
    si = si_ref[...]
    di = di_ref[...]
    sj = sj_ref[...]
    dj = dj_ref[...]
    zs = jnp.maximum(edm2 + es2, c22 + es022_ref[...])
    pa = (si - sj) * (si - dj)
    pb = (di - sj) * (di - dj)
    p = jnp.where(pa * pb == 0.0, jnp.exp2(zs), jnp.float32(0.0))
    acc = jnp.dot(p.astype(jnp.bfloat16), hbf_ref[...],
                  preferred_element_type=jnp.float32)
    s = acc[:, C:C + 1]
    out_ref[...] = acc[:, :C] * (1.0 / s) + b_ref[...]


def kernel(x, edge_index, W, a_src, a_dst, b):
    E, _ = x.shape
    C = W.shape[1]
    blk = 512
    srcf = edge_index[0].astype(jnp.float32)
    dstf = edge_index[1].astype(jnp.float32)
    out = pl.pallas_call(
        _gat_kernel,
        grid=(E // blk,),
        in_specs=[
            pl.BlockSpec((E, x.shape[1]), lambda i: (0, 0)),
            pl.BlockSpec((x.shape[1], C), lambda i: (0, 0)),
            pl.BlockSpec((1, C), lambda i: (0, 0)),
            pl.BlockSpec((C, 1), lambda i: (0, 0)),
            pl.BlockSpec((blk, 1), lambda i: (i, 0)),
            pl.BlockSpec((blk, 1), lambda i: (i, 0)),
            pl.BlockSpec((1, E), lambda i: (0, 0)),
            pl.BlockSpec((1, E), lambda i: (0, 0)),
            pl.BlockSpec((1, C), lambda i: (0, 0)),
        ],
        out_specs=pl.BlockSpec((blk, C), lambda i: (i, 0)),
        out_shape=jax.ShapeDtypeStruct((E, C), jnp.float32),
        scratch_shapes=[
            pltpu.VMEM((E, C), jnp.float32),
            pltpu.VMEM((E, C + 128), jnp.bfloat16),
            pltpu.VMEM((1, E), jnp.float32),
            pltpu.VMEM((1, E), jnp.float32),
        ],
    )(x, W, a_src.reshape(1, C), a_dst.reshape(C, 1),
      srcf.reshape(E, 1), dstf.reshape(E, 1),
      srcf.reshape(1, E), dstf.reshape(1, E),
      b.reshape(1, C))
    return out


# speedup vs baseline: 2.4180x; 1.1817x over previous
import jax
import jax.numpy as jnp
from jax.experimental import pallas as pl
from jax.experimental.pallas import tpu as pltpu

_LOG2E = 1.4426950408889634


def _gat_kernel(x_ref, w_ref, asrc_ref, adst_ref, si_ref, di_ref, sj_ref,
                dj_ref, b_ref, out_ref, h_ref, hbf_ref, es2_ref, es022_ref):
    i = pl.program_id(0)
    blk = out_ref.shape[0]
    C = out_ref.shape[1]

    @pl.when(i == 0)
    def _init():
        h = jnp.dot(x_ref[...], w_ref[...], preferred_element_type=jnp.float32)
        h_ref[...] = h
        hbf_ref[:, :C] = h.astype(jnp.bfloat16)
        hbf_ref[:, C:] = jnp.ones((h.shape[0], 128), jnp.bfloat16)
        es = jax.lax.dot_general(asrc_ref[...], h, (((1,), (1,)), ((), ())),
                                 preferred_element_type=jnp.float32)
        es2_ref[...] = _LOG2E * es
        es022_ref[...] = (0.2 * _LOG2E) * es

    hb = h_ref[pl.ds(i * blk, blk), :]
    ed = jnp.dot(hb, adst_ref[...], preferred_element_type=jnp.float32)
    es2 = es2_ref[...]
    smax2 = jnp.max(es2)
    q2 = _LOG2E * ed + smax2
    mi2 = jnp.maximum(q2, 0.2 * q2)
    edm2 = _LOG2E * ed - mi2
    c22 = (0.2 * _LOG2E) * ed - mi2
    si = si_ref[...]
    di = di_ref[...]
    sj = sj_ref[...]
    dj = dj_ref[...]
    zs = jnp.maximum(edm2 + es2, c22 + es022_ref[...])
    conn = ((si == sj) | (si == dj)) | ((di == sj) | (di == dj))
    p = jnp.where(conn, jnp.exp2(zs), jnp.float32(0.0))
    acc = jnp.dot(p.astype(jnp.bfloat16), hbf_ref[...],
                  preferred_element_type=jnp.float32)
    s = acc[:, C:C + 1]
    out_ref[...] = acc[:, :C] * (1.0 / s) + b_ref[...]


def kernel(x, edge_index, W, a_src, a_dst, b):
    E, _ = x.shape
    C = W.shape[1]
    blk = 512
    srcf = edge_index[0].astype(jnp.int16)
    dstf = edge_index[1].astype(jnp.int16)
    out = pl.pallas_call(
        _gat_kernel,
        grid=(E // blk,),
        in_specs=[
            pl.BlockSpec((E, x.shape[1]), lambda i: (0, 0)),
            pl.BlockSpec((x.shape[1], C), lambda i: (0, 0)),
            pl.BlockSpec((1, C), lambda i: (0, 0)),
            pl.BlockSpec((C, 1), lambda i: (0, 0)),
            pl.BlockSpec((blk, 1), lambda i: (i, 0)),
            pl.BlockSpec((blk, 1), lambda i: (i, 0)),
            pl.BlockSpec((1, E), lambda i: (0, 0)),
            pl.BlockSpec((1, E), lambda i: (0, 0)),
            pl.BlockSpec((1, C), lambda i: (0, 0)),
        ],
        out_specs=pl.BlockSpec((blk, C), lambda i: (i, 0)),
        out_shape=jax.ShapeDtypeStruct((E, C), jnp.float32),
        scratch_shapes=[
            pltpu.VMEM((E, C), jnp.float32),
            pltpu.VMEM((E, C + 128), jnp.bfloat16),
            pltpu.VMEM((1, E), jnp.float32),
            pltpu.VMEM((1, E), jnp.float32),
        ],
    )(x, W, a_src.reshape(1, C), a_dst.reshape(C, 1),
      srcf.reshape(E, 1), dstf.reshape(E, 1),
      srcf.reshape(1, E), dstf.reshape(1, E),
      b.reshape(1, C))
    return out


# speedup vs baseline: 2.6789x; 1.1079x over previous
import jax
import jax.numpy as jnp
from jax.experimental import pallas as pl
from jax.experimental.pallas import tpu as pltpu

_LOG2E = 1.4426950408889634


def _gat_kernel(x_ref, w_ref, asrc_ref, adst_ref, ei_ref, b_ref, out_ref,
                hbf_ref, es2_ref, es022_ref, edcol_ref, eirow_ref, eicol_ref):
    i = pl.program_id(0)
    blk = out_ref.shape[0]
    C = out_ref.shape[1]

    @pl.when(i == 0)
    def _init():
        h = jnp.dot(x_ref[...], w_ref[...], preferred_element_type=jnp.float32)
        hbf_ref[:, :C] = h.astype(jnp.bfloat16)
        hbf_ref[:, C:] = jnp.ones((h.shape[0], 128), jnp.bfloat16)
        es = jax.lax.dot_general(asrc_ref[...], h, (((1,), (1,)), ((), ())),
                                 preferred_element_type=jnp.float32)
        es2_ref[...] = _LOG2E * es
        es022_ref[...] = (0.2 * _LOG2E) * es
        edcol_ref[...] = jnp.dot(h, adst_ref[...],
                                 preferred_element_type=jnp.float32)
        ei = ei_ref[...]
        eirow_ref[...] = ei.astype(jnp.int16)
        eicol_ref[...] = jnp.transpose(ei.astype(jnp.float32)).astype(jnp.int16)

    ed = edcol_ref[pl.ds(i * blk, blk), :]
    es2 = es2_ref[...]
    smax2 = jnp.max(es2)
    q2 = _LOG2E * ed + smax2
    mi2 = jnp.maximum(q2, 0.2 * q2)
    edm2 = _LOG2E * ed - mi2
    c22 = (0.2 * _LOG2E) * ed - mi2
    si = eicol_ref[pl.ds(i * blk, blk), 0:1]
    di = eicol_ref[pl.ds(i * blk, blk), 1:2]
    sj = eirow_ref[0:1, :]
    dj = eirow_ref[1:2, :]
    zs = jnp.maximum(edm2 + es2, c22 + es022_ref[...])
    conn = ((si == sj) | (si == dj)) | ((di == sj) | (di == dj))
    p = jnp.where(conn, jnp.exp2(zs), jnp.float32(0.0))
    acc = jnp.dot(p.astype(jnp.bfloat16), hbf_ref[...],
                  preferred_element_type=jnp.float32)
    s = acc[:, C:C + 1]
    out_ref[...] = acc[:, :C] * (1.0 / s) + b_ref[...]


def kernel(x, edge_index, W, a_src, a_dst, b):
    E, _ = x.shape
    C = W.shape[1]
    blk = 512
    out = pl.pallas_call(
        _gat_kernel,
        grid=(E // blk,),
        in_specs=[
            pl.BlockSpec((E, x.shape[1]), lambda i: (0, 0)),
            pl.BlockSpec((x.shape[1], C), lambda i: (0, 0)),
            pl.BlockSpec((1, C), lambda i: (0, 0)),
            pl.BlockSpec((C, 1), lambda i: (0, 0)),
            pl.BlockSpec((2, E), lambda i: (0, 0)),
            pl.BlockSpec((1, C), lambda i: (0, 0)),
        ],
        out_specs=pl.BlockSpec((blk, C), lambda i: (i, 0)),
        out_shape=jax.ShapeDtypeStruct((E, C), jnp.float32),
        scratch_shapes=[
            pltpu.VMEM((E, C + 128), jnp.bfloat16),
            pltpu.VMEM((1, E), jnp.float32),
            pltpu.VMEM((1, E), jnp.float32),
            pltpu.VMEM((E, 1), jnp.float32),
            pltpu.VMEM((2, E), jnp.int16),
            pltpu.VMEM((E, 2), jnp.int16),
        ],
    )(x, W, a_src.reshape(1, C), a_dst.reshape(C, 1), edge_index,
      b.reshape(1, C))
    return out


# speedup vs baseline: 2.9835x; 1.1137x over previous
import jax
import jax.numpy as jnp
from jax.experimental import pallas as pl
from jax.experimental.pallas import tpu as pltpu

_LOG2E = 1.4426950408889634


def _gat_kernel(x_ref, w_ref, asrc_ref, adst_ref, ei_ref, b_ref, out_ref,
                hbf_ref, es2_ref, es022_ref, edcol_ref, eirow_ref, eicol_ref):
    i = pl.program_id(0)
    blk = out_ref.shape[0]
    C = out_ref.shape[1]

    @pl.when(i == 0)
    def _init():
        h = jnp.dot(x_ref[...], w_ref[...], preferred_element_type=jnp.float32)
        hbf_ref[:, :C] = h.astype(jnp.bfloat16)
        hbf_ref[:, C:] = jnp.ones((h.shape[0], 128), jnp.bfloat16)
        es = jax.lax.dot_general(asrc_ref[...], h, (((1,), (1,)), ((), ())),
                                 preferred_element_type=jnp.float32)
        es2_ref[...] = _LOG2E * es
        es022_ref[...] = (0.2 * _LOG2E) * es
        edcol_ref[...] = jnp.dot(h, adst_ref[...],
                                 preferred_element_type=jnp.float32)
        ei = ei_ref[...]
        eirow_ref[...] = ei.astype(jnp.int16)
        eicol_ref[...] = jnp.transpose(ei.astype(jnp.float32)).astype(jnp.int16)

    ed = edcol_ref[pl.ds(i * blk, blk), :]
    es2 = es2_ref[...]
    smax2 = jnp.max(es2)
    q2 = _LOG2E * ed + smax2
    mi2 = jnp.maximum(q2, 0.2 * q2)
    edm2 = _LOG2E * ed - mi2
    c22 = (0.2 * _LOG2E) * ed - mi2
    si = eicol_ref[pl.ds(i * blk, blk), 0:1]
    di = eicol_ref[pl.ds(i * blk, blk), 1:2]
    sj = eirow_ref[0:1, :]
    dj = eirow_ref[1:2, :]
    zs = jnp.maximum(edm2 + es2, c22 + es022_ref[...])
    conn = ((si == sj) | (si == dj)) | ((di == sj) | (di == dj))
    p = jnp.where(conn, jnp.exp2(zs).astype(jnp.bfloat16), jnp.bfloat16(0.0))
    acc = jnp.dot(p, hbf_ref[...],
                  preferred_element_type=jnp.float32)
    s = acc[:, C:C + 1]
    out_ref[...] = acc[:, :C] * (1.0 / s) + b_ref[...]


def kernel(x, edge_index, W, a_src, a_dst, b):
    E, _ = x.shape
    C = W.shape[1]
    blk = 512
    out = pl.pallas_call(
        _gat_kernel,
        grid=(E // blk,),
        in_specs=[
            pl.BlockSpec((E, x.shape[1]), lambda i: (0, 0)),
            pl.BlockSpec((x.shape[1], C), lambda i: (0, 0)),
            pl.BlockSpec((1, C), lambda i: (0, 0)),
            pl.BlockSpec((C, 1), lambda i: (0, 0)),
            pl.BlockSpec((2, E), lambda i: (0, 0)),
            pl.BlockSpec((1, C), lambda i: (0, 0)),
        ],
        out_specs=pl.BlockSpec((blk, C), lambda i: (i, 0)),
        out_shape=jax.ShapeDtypeStruct((E, C), jnp.float32),
        scratch_shapes=[
            pltpu.VMEM((E, C + 128), jnp.bfloat16),
            pltpu.VMEM((1, E), jnp.float32),
            pltpu.VMEM((1, E), jnp.float32),
            pltpu.VMEM((E, 1), jnp.float32),
            pltpu.VMEM((2, E), jnp.int16),
            pltpu.VMEM((E, 2), jnp.int16),
        ],
    )(x, W, a_src.reshape(1, C), a_dst.reshape(C, 1), edge_index,
      b.reshape(1, C))
    return out
